# Initial kernel scaffold; baseline (speedup 1.0000x reference)
import functools

import jax
import jax.numpy as jnp
from jax import lax
from jax.experimental import pallas as pl
from jax.experimental.pallas import tpu as pltpu
from jax.experimental.pallas import tpu_sc as plsc

F = 26
VOCAB = 100000
D = 32
B = 16384

TOTAL = B * F
NC = 2
NS = 16
NW = NC * NS
PER_W = TOTAL // NW
CHUNK = 1664
NCHUNK = PER_W // CHUNK

_mesh = plsc.VectorSubcoreMesh(core_axis_name="c", subcore_axis_name="s")


@functools.partial(
    pl.kernel,
    mesh=_mesh,
    out_type=jax.ShapeDtypeStruct((TOTAL, D), jnp.float32),
    scratch_types=[
        pltpu.VMEM((CHUNK,), jnp.int32),
        pltpu.VMEM((CHUNK, D), jnp.float32),
        pltpu.SemaphoreType.DMA,
    ],
)
def _gather_kernel(x_hbm, tables_hbm, out_hbm, idx_v, rows_v, sem):
    wid = lax.axis_index("s") * NC + lax.axis_index("c")
    base = wid * PER_W

    def chunk_body(ci, carry):
        cbase = base + ci * CHUNK
        pltpu.sync_copy(x_hbm.at[pl.ds(cbase, CHUNK)], idx_v)

        def grp(g, c):
            j = g * 16
            lane = j + lax.iota(jnp.int32, 16)
            off = lax.rem(lane, F) * VOCAB
            idx_v[pl.ds(j, 16)] = idx_v[pl.ds(j, 16)] + off
            return c

        lax.fori_loop(0, CHUNK // 16, grp, 0)

        pltpu.async_copy(tables_hbm.at[idx_v], rows_v, sem).wait()
        pltpu.sync_copy(rows_v, out_hbm.at[pl.ds(cbase, CHUNK)])
        return carry

    lax.fori_loop(0, NCHUNK, chunk_body, 0)


def kernel(x, tables):
    x_flat = x.reshape(TOTAL)
    tables_flat = tables.reshape(F * VOCAB, D)
    out = _gather_kernel(x_flat, tables_flat)
    return out.reshape(B, F, D)


# speedup vs baseline: 1.1451x; 1.1451x over previous
import functools

import jax
import jax.numpy as jnp
from jax import lax
from jax.experimental import pallas as pl
from jax.experimental.pallas import tpu as pltpu
from jax.experimental.pallas import tpu_sc as plsc

F = 26
VOCAB = 100000
D = 32
B = 16384

TOTAL = B * F
NC = 2
NS = 16
NW = NC * NS
PER_W = TOTAL // NW
CHUNK = 1664
NCHUNK = PER_W // CHUNK

_mesh = plsc.VectorSubcoreMesh(core_axis_name="c", subcore_axis_name="s")


@functools.partial(
    pl.kernel,
    mesh=_mesh,
    out_type=jax.ShapeDtypeStruct((TOTAL, D), jnp.float32),
    scratch_types=[
        pltpu.VMEM((CHUNK,), jnp.int32),
        pltpu.VMEM((CHUNK, D), jnp.float32),
        pltpu.SemaphoreType.DMA,
    ],
    compiler_params=pltpu.CompilerParams(use_tc_tiling_on_sc=False),
)
def _gather_kernel(x_hbm, tables_hbm, out_hbm, idx_v, rows_v, sem):
    wid = lax.axis_index("s") * NC + lax.axis_index("c")
    base = wid * PER_W

    def chunk_body(ci, carry):
        cbase = base + ci * CHUNK
        pltpu.sync_copy(x_hbm.at[pl.ds(cbase, CHUNK)], idx_v)

        def grp(g, c):
            j = g * 16
            lane = j + lax.iota(jnp.int32, 16)
            off = lax.rem(lane, F) * VOCAB
            idx_v[pl.ds(j, 16)] = idx_v[pl.ds(j, 16)] + off
            return c

        lax.fori_loop(0, CHUNK // 16, grp, 0)

        pltpu.async_copy(tables_hbm.at[idx_v], rows_v, sem).wait()
        pltpu.sync_copy(rows_v, out_hbm.at[pl.ds(cbase, CHUNK)])
        return carry

    lax.fori_loop(0, NCHUNK, chunk_body, 0)


def kernel(x, tables):
    x_flat = x.reshape(TOTAL)
    tables_flat = tables.reshape(F * VOCAB, D)
    out = _gather_kernel(x_flat, tables_flat)
    return out.reshape(B, F, D)


# speedup vs baseline: 1.1501x; 1.0044x over previous
import functools

import jax
import jax.numpy as jnp
from jax import lax
from jax.experimental import pallas as pl
from jax.experimental.pallas import tpu as pltpu
from jax.experimental.pallas import tpu_sc as plsc

F = 26
VOCAB = 100000
D = 32
B = 16384

TOTAL = B * F
NC = 2
NS = 16
NW = NC * NS
PER_W = TOTAL // NW
CHUNK = 1664
NCHUNK = PER_W // CHUNK
PERIOD = 208

_mesh = plsc.VectorSubcoreMesh(core_axis_name="c", subcore_axis_name="s")


@functools.partial(
    pl.kernel,
    mesh=_mesh,
    out_type=jax.ShapeDtypeStruct((TOTAL, D), jnp.float32),
    scratch_types=[
        pltpu.VMEM((NCHUNK, CHUNK), jnp.int32),
        pltpu.VMEM((PERIOD,), jnp.int32),
        pltpu.VMEM((CHUNK, D), jnp.float32),
        pltpu.VMEM((CHUNK, D), jnp.float32),
        pltpu.SemaphoreType.DMA,
        pltpu.SemaphoreType.DMA,
        pltpu.SemaphoreType.DMA,
        pltpu.SemaphoreType.DMA,
    ],
    compiler_params=pltpu.CompilerParams(use_tc_tiling_on_sc=False),
)
def _gather_kernel(x_hbm, tables_hbm, out_hbm, idx_v, off_v, buf0, buf1,
                   g0, g1, w0, w1):
    wid = lax.axis_index("s") * NC + lax.axis_index("c")
    base = wid * PER_W

    pltpu.sync_copy(x_hbm.at[wid], idx_v)

    for k in range(PERIOD // 16):
        lane = k * 16 + lax.iota(jnp.int32, 16)
        off_v[pl.ds(k * 16, 16)] = lax.rem(lane, F) * VOCAB

    for c in range(NCHUNK):
        def body(r, carry, c=c):
            jb = r * PERIOD
            for k in range(PERIOD // 16):
                sl = pl.ds(jb + k * 16, 16)
                osl = pl.ds(k * 16, 16)
                idx_v[c, sl] = idx_v[c, sl] + off_v[osl]
            return carry

        lax.fori_loop(0, CHUNK // PERIOD, body, 0)

    bufs = [buf0, buf1]
    gsems = [g0, g1]
    wsems = [w0, w1]
    gd = [None] * NCHUNK
    wd = [None] * NCHUNK

    gd[0] = pltpu.async_copy(tables_hbm.at[idx_v.at[0]], bufs[0], gsems[0])
    for c in range(NCHUNK):
        if c + 1 < NCHUNK:
            if c >= 1:
                wd[c - 1].wait()
            gd[c + 1] = pltpu.async_copy(
                tables_hbm.at[idx_v.at[c + 1]], bufs[(c + 1) % 2],
                gsems[(c + 1) % 2])
        gd[c].wait()
        wd[c] = pltpu.async_copy(
            bufs[c % 2], out_hbm.at[pl.ds(base + c * CHUNK, CHUNK)],
            wsems[c % 2])
    wd[NCHUNK - 2].wait()
    wd[NCHUNK - 1].wait()


def kernel(x, tables):
    x_grp = x.reshape(NW, NCHUNK, CHUNK)
    tables_flat = tables.reshape(F * VOCAB, D)
    out = _gather_kernel(x_grp, tables_flat)
    return out.reshape(B, F, D)


# speedup vs baseline: 1.1510x; 1.0008x over previous
import functools

import jax
import jax.numpy as jnp
from jax import lax
from jax.experimental import pallas as pl
from jax.experimental.pallas import tpu as pltpu
from jax.experimental.pallas import tpu_sc as plsc

F = 26
VOCAB = 100000
D = 32
B = 16384

TOTAL = B * F
NC = 2
NS = 16
NW = NC * NS
PER_W = TOTAL // NW
CHUNK = 1664
NCHUNK = PER_W // CHUNK
NSUB = 4
SUB = CHUNK // NSUB
PERIOD = 208

_mesh = plsc.VectorSubcoreMesh(core_axis_name="c", subcore_axis_name="s")


@functools.partial(
    pl.kernel,
    mesh=_mesh,
    out_type=jax.ShapeDtypeStruct((TOTAL, D), jnp.float32),
    scratch_types=[
        pltpu.VMEM((NCHUNK * NSUB, SUB), jnp.int32),
        pltpu.VMEM((PERIOD,), jnp.int32),
        pltpu.VMEM((CHUNK, D), jnp.float32),
        pltpu.VMEM((CHUNK, D), jnp.float32),
        pltpu.SemaphoreType.DMA,
        pltpu.SemaphoreType.DMA,
        pltpu.SemaphoreType.DMA,
        pltpu.SemaphoreType.DMA,
    ],
    compiler_params=pltpu.CompilerParams(use_tc_tiling_on_sc=False),
)
def _gather_kernel(x_hbm, tables_hbm, out_hbm, idx_v, off_v, buf0, buf1,
                   g0, g1, w0, w1):
    wid = lax.axis_index("s") * NC + lax.axis_index("c")
    base = wid * PER_W

    pltpu.sync_copy(x_hbm.at[wid], idx_v)

    for k in range(PERIOD // 16):
        lane = k * 16 + lax.iota(jnp.int32, 16)
        off_v[pl.ds(k * 16, 16)] = lax.rem(lane, F) * VOCAB

    def row_body(ci, carry):
        def per_body(r, carry2):
            jb = r * PERIOD
            for k in range(PERIOD // 16):
                sl = pl.ds(jb + k * 16, 16)
                osl = pl.ds(k * 16, 16)
                idx_v[ci, sl] = idx_v[ci, sl] + off_v[osl]
            return carry2

        return lax.fori_loop(0, SUB // PERIOD, per_body, carry)

    lax.fori_loop(0, NCHUNK * NSUB, row_body, 0)

    bufs = [buf0, buf1]
    gsems = [g0, g1]
    wsems = [w0, w1]
    wd = [None] * NCHUNK

    def fire(c):
        return [
            pltpu.async_copy(
                tables_hbm.at[idx_v.at[c * NSUB + s]],
                bufs[c % 2].at[pl.ds(s * SUB, SUB)], gsems[c % 2])
            for s in range(NSUB)
        ]

    gd = [None] * NCHUNK
    gd[0] = fire(0)
    for c in range(NCHUNK):
        if c + 1 < NCHUNK:
            if c >= 1:
                wd[c - 1].wait()
            gd[c + 1] = fire(c + 1)
        for d in gd[c]:
            d.wait()
        wd[c] = pltpu.async_copy(
            bufs[c % 2], out_hbm.at[pl.ds(base + c * CHUNK, CHUNK)],
            wsems[c % 2])
    wd[NCHUNK - 2].wait()
    wd[NCHUNK - 1].wait()


def kernel(x, tables):
    x_grp = x.reshape(NW, NCHUNK * NSUB, SUB)
    tables_flat = tables.reshape(F * VOCAB, D)
    out = _gather_kernel(x_grp, tables_flat)
    return out.reshape(B, F, D)


# speedup vs baseline: 4.3790x; 3.8046x over previous
import functools

import jax
import jax.numpy as jnp
from jax import lax
from jax.experimental import pallas as pl
from jax.experimental.pallas import tpu as pltpu
from jax.experimental.pallas import tpu_sc as plsc

F = 26
VOCAB = 100000
D = 32
B = 16384

NC = 2
NS = 16
NW = NC * NS
PAIRS = F * D
PER_W = PAIRS // NW
HALF = B // 2

_mesh = plsc.VectorSubcoreMesh(core_axis_name="c", subcore_axis_name="s")


@functools.partial(
    pl.kernel,
    mesh=_mesh,
    out_type=jax.ShapeDtypeStruct((F, D, B), jnp.float32),
    scratch_types=[
        pltpu.VMEM((VOCAB,), jnp.float32),
        pltpu.VMEM((HALF,), jnp.int32),
        pltpu.VMEM((HALF,), jnp.float32),
        pltpu.SemaphoreType.DMA,
    ],
    compiler_params=pltpu.CompilerParams(needs_layout_passes=False),
)
def _lookup_kernel(x_hbm, tables_hbm, out_hbm, row_v, idx_v, out_v, wsem):
    wid = lax.axis_index("s") * NC + lax.axis_index("c")

    def pair_body(j, carry):
        p = wid * PER_W + j
        f = lax.shift_right_logical(p, 5)
        d = lax.bitwise_and(p, D - 1)

        pltpu.sync_copy(tables_hbm.at[f, d], row_v)

        def half_body(h, carry2):
            pltpu.sync_copy(x_hbm.at[f, pl.ds(h * HALF, HALF)], idx_v)

            def grp(g, carry3):
                sl = pl.ds(g * 16, 16)
                out_v[sl] = plsc.load_gather(row_v, [idx_v[sl]])
                return carry3

            lax.fori_loop(0, HALF // 16, grp, 0)
            pltpu.async_copy(
                out_v, out_hbm.at[f, d, pl.ds(h * HALF, HALF)], wsem).wait()
            return carry2

        lax.fori_loop(0, 2, half_body, 0)
        return carry

    lax.fori_loop(0, PER_W, pair_body, 0)


def kernel(x, tables):
    x_t = jnp.swapaxes(x, 0, 1)
    tables_t = jnp.transpose(tables, (0, 2, 1))
    out = _lookup_kernel(x_t, tables_t)
    return jnp.transpose(out, (2, 0, 1))


# speedup vs baseline: 5.3178x; 1.2144x over previous
import functools

import jax
import jax.numpy as jnp
from jax import lax
from jax.experimental import pallas as pl
from jax.experimental.pallas import tpu as pltpu
from jax.experimental.pallas import tpu_sc as plsc

F = 26
VOCAB = 100000
D = 32
B = 16384

NC = 2
NS = 16
NW = NC * NS
PAIRS = F * D
PER_W = PAIRS // NW
HALF = B // 2

_mesh = plsc.VectorSubcoreMesh(core_axis_name="c", subcore_axis_name="s")


@functools.partial(
    pl.kernel,
    mesh=_mesh,
    out_type=jax.ShapeDtypeStruct((F, D, B), jnp.float32),
    scratch_types=[
        pltpu.VMEM((VOCAB,), jnp.float32),
        pltpu.VMEM((HALF,), jnp.int32),
        pltpu.VMEM((HALF,), jnp.float32),
        pltpu.VMEM((HALF,), jnp.float32),
        pltpu.SemaphoreType.DMA,
        pltpu.SemaphoreType.DMA,
    ],
    compiler_params=pltpu.CompilerParams(needs_layout_passes=False),
)
def _lookup_kernel(x_hbm, tables_hbm, out_hbm, row_v, idx_v, out_v0, out_v1,
                   w0, w1):
    wid = lax.axis_index("s") * NC + lax.axis_index("c")
    outs = [out_v0, out_v1]
    wsems = [w0, w1]

    def pair_body(j, carry):
        p = wid * PER_W + j
        f = lax.shift_right_logical(p, 5)
        d = lax.bitwise_and(p, D - 1)

        pltpu.sync_copy(tables_hbm.at[f, d], row_v)

        for h in range(2):
            out_v = outs[h]
            pltpu.sync_copy(x_hbm.at[f, pl.ds(h * HALF, HALF)], idx_v)

            @pl.when(j > 0)
            def _(out_v=out_v, h=h):
                pltpu.make_async_copy(
                    out_v, out_hbm.at[0, 0, pl.ds(h * HALF, HALF)],
                    wsems[h]).wait()

            def grp(g, carry3, out_v=out_v):
                for u in range(8):
                    sl = pl.ds(g * 128 + u * 16, 16)
                    out_v[sl] = plsc.load_gather(row_v, [idx_v[sl]])
                return carry3

            lax.fori_loop(0, HALF // 128, grp, 0)
            pltpu.async_copy(
                out_v, out_hbm.at[f, d, pl.ds(h * HALF, HALF)], wsems[h])

        return carry

    lax.fori_loop(0, PER_W, pair_body, 0)
    for h in range(2):
        pltpu.make_async_copy(
            outs[h], out_hbm.at[0, 0, pl.ds(h * HALF, HALF)],
            wsems[h]).wait()


def kernel(x, tables):
    x_t = jnp.swapaxes(x, 0, 1)
    tables_t = jnp.transpose(tables, (0, 2, 1))
    out = _lookup_kernel(x_t, tables_t)
    return jnp.transpose(out, (2, 0, 1))


# speedup vs baseline: 6.9238x; 1.3020x over previous
import functools

import jax
import jax.numpy as jnp
from jax import lax
from jax.experimental import pallas as pl
from jax.experimental.pallas import tpu as pltpu
from jax.experimental.pallas import tpu_sc as plsc

F = 26
VOCAB = 100000
D = 32
B = 16384

NC = 2
NS = 16
NW = NC * NS
PAIRS = F * D
PER_W = PAIRS // NW
HALF = B // 2

_mesh = plsc.VectorSubcoreMesh(core_axis_name="c", subcore_axis_name="s")


@functools.partial(
    pl.kernel,
    mesh=_mesh,
    out_type=jax.ShapeDtypeStruct((F, D, B), jnp.float32),
    scratch_types=[
        pltpu.VMEM((VOCAB,), jnp.float32),
        pltpu.VMEM((HALF,), jnp.int32),
        pltpu.VMEM((HALF,), jnp.float32),
        pltpu.VMEM((HALF,), jnp.float32),
        pltpu.SemaphoreType.DMA,
        pltpu.SemaphoreType.DMA,
    ],
    compiler_params=pltpu.CompilerParams(needs_layout_passes=False),
)
def _lookup_kernel(x_hbm, tables_hbm, out_hbm, row_v, idx_v, out_v0, out_v1,
                   w0, w1):
    wid = lax.axis_index("s") * NC + lax.axis_index("c")
    outs = [out_v0, out_v1]
    wsems = [w0, w1]

    def pair_body(j, carry):
        p = wid * PER_W + j
        f = lax.shift_right_logical(p, 5)
        d = lax.bitwise_and(p, D - 1)

        pltpu.sync_copy(tables_hbm.at[f, d], row_v)

        for h in range(2):
            out_v = outs[h]
            pltpu.sync_copy(x_hbm.at[f, pl.ds(h * HALF, HALF)], idx_v)

            @pl.when(j > 0)
            def _(out_v=out_v, h=h):
                pltpu.make_async_copy(
                    out_v, out_hbm.at[0, 0, pl.ds(h * HALF, HALF)],
                    wsems[h]).wait()

            @plsc.parallel_loop(0, HALF // 128, 1, unroll=2)
            def _(g, out_v=out_v):
                for u in range(8):
                    sl = pl.ds(g * 128 + u * 16, 16)
                    out_v[sl] = plsc.load_gather(row_v, [idx_v[sl]])
            pltpu.async_copy(
                out_v, out_hbm.at[f, d, pl.ds(h * HALF, HALF)], wsems[h])

        return carry

    lax.fori_loop(0, PER_W, pair_body, 0)
    for h in range(2):
        pltpu.make_async_copy(
            outs[h], out_hbm.at[0, 0, pl.ds(h * HALF, HALF)],
            wsems[h]).wait()


def kernel(x, tables):
    x_t = jnp.swapaxes(x, 0, 1)
    tables_t = jnp.transpose(tables, (0, 2, 1))
    out = _lookup_kernel(x_t, tables_t)
    return jnp.transpose(out, (2, 0, 1))


# speedup vs baseline: 9.7740x; 1.4116x over previous
import functools

import jax
import jax.numpy as jnp
from jax import lax
from jax.experimental import pallas as pl
from jax.experimental.pallas import tpu as pltpu
from jax.experimental.pallas import tpu_sc as plsc

F = 26
VOCAB = 100000
D = 32
B = 16384

NC = 2
NS = 16
NW = NC * NS
PAIRS = F * D
PER_W = PAIRS // NW
NBLK = PER_W // 2

VTH = 33408
VSZ2 = VOCAB - 2 * VTH
VBODY = VSZ2 - 32

_mesh = plsc.VectorSubcoreMesh(core_axis_name="c", subcore_axis_name="s")


@functools.partial(
    pl.kernel,
    mesh=_mesh,
    out_type=jax.ShapeDtypeStruct((F, D, B), jnp.float32),
    scratch_types=[
        pltpu.VMEM((VTH,), jnp.float32),
        pltpu.VMEM((VTH,), jnp.float32),
        pltpu.VMEM((B,), jnp.int32),
        pltpu.VMEM((B,), jnp.float32),
        pltpu.VMEM((B,), jnp.float32),
        pltpu.SemaphoreType.DMA,
        pltpu.SemaphoreType.DMA,
        pltpu.SemaphoreType.DMA,
        pltpu.SemaphoreType.DMA,
    ],
    compiler_params=pltpu.CompilerParams(needs_layout_passes=False),
)
def _lookup_kernel(x_hbm, tables_hbm, tail_hbm, out_hbm, row0, row1, idx_v,
                   ob0, ob1, r0, r1, w0, w1):
    wid = lax.axis_index("s") * NC + lax.axis_index("c")
    rows = [row0, row1]
    rsems = [r0, r1]
    outs = [ob0, ob1]
    wsems = [w0, w1]

    def fd(j):
        p = wid * PER_W + j
        return lax.shift_right_logical(p, 5), lax.bitwise_and(p, D - 1), p

    def load_parts(j, v, par):
        f, d, p = fd(j)
        if v < 2:
            return [(tables_hbm.at[f, d, pl.ds(v * VTH, VTH)],
                     rows[par].at[pl.ds(0, VTH)])]
        return [(tables_hbm.at[f, d, pl.ds(2 * VTH, VBODY)],
                 rows[par].at[pl.ds(0, VBODY)]),
                (tail_hbm.at[p], rows[par].at[pl.ds(VBODY, 128)])]

    def start_load(j, v, par):
        for src, dst in load_parts(j, v, par):
            pltpu.async_copy(src, dst, rsems[par])

    def wait_load(j, v, par):
        for src, dst in load_parts(j, v, par):
            pltpu.make_async_copy(src, dst, rsems[par]).wait()

    start_load(0, 0, 0)
    iota = lax.iota(jnp.int32, 16)

    def block(i, carry):
        for k in range(6):
            half = k // 3
            v = k % 3
            par = k % 2
            jj = 2 * i + half
            f, d, _ = fd(jj)

            if v == 0:
                fprev, _, _ = fd(lax.max(jj - 1, 0))

                @pl.when((jj == 0) | (f != fprev))
                def _(f=f):
                    pltpu.sync_copy(x_hbm.at[f], idx_v)

                @pl.when(jj >= 2)
                def _(half=half):
                    pltpu.make_async_copy(
                        outs[half], out_hbm.at[0, 0], wsems[half]).wait()

            if k < 5:
                nk = k + 1
                start_load(2 * i + nk // 3, nk % 3, nk % 2)
            else:
                @pl.when(i < NBLK - 1)
                def _():
                    start_load(2 * i + 2, 0, 0)

            wait_load(jj, v, par)
            row_v = rows[par]
            ob = outs[half]
            vsz = VTH if v < 2 else VSZ2

            @plsc.parallel_loop(0, B // 128, 1, unroll=2)
            def _(g, v=v, vsz=vsz, row_v=row_v, ob=ob):
                for u in range(8):
                    base = g * 128 + u * 16
                    sl = pl.ds(base, 16)
                    iv = idx_v[sl]
                    local = iv if v == 0 else iv - v * VTH
                    mask = plsc.bitcast(local, jnp.uint32) < jnp.uint32(vsz)
                    gv = plsc.load_gather(row_v, [local], mask=mask)
                    plsc.store_scatter(ob, [base + iota], gv, mask=mask)

            if v == 2:
                pltpu.async_copy(ob, out_hbm.at[f, d], wsems[half])
        return carry

    lax.fori_loop(0, NBLK, block, 0)
    for half in range(2):
        pltpu.make_async_copy(
            outs[half], out_hbm.at[0, 0], wsems[half]).wait()


def kernel(x, tables):
    x_t = jnp.swapaxes(x, 0, 1)
    tables_t = jnp.transpose(tables, (0, 2, 1))
    tail = jnp.transpose(tables[:, 2 * VTH + VBODY:, :], (0, 2, 1))
    tail = jnp.pad(tail.reshape(PAIRS, 32), ((0, 0), (0, 96)))
    out = _lookup_kernel(x_t, tables_t, tail)
    return jnp.transpose(out, (2, 0, 1))
